# Initial kernel scaffold; baseline (speedup 1.0000x reference)
import jax
import jax.numpy as jnp
from jax.experimental import pallas as pl


def kernel(x, edge_index, W1, a1s, a1d, b1, W2, a2s, a2d, b2, W3, a3s, a3d, b3, W4, a4s, a4d, b4):
    raise NotImplementedError("write your pallas kernel here")


# speedup vs baseline: 43.8058x; 43.8058x over previous
import functools

import jax
import jax.numpy as jnp
from jax import lax
from jax.experimental import pallas as pl
from jax.experimental.pallas import tpu as pltpu
from jax.experimental.pallas import tpu_sc as plsc

_N = 10000
_E = 320000
_D = 128
_H = 8
_HD = 8
_C = 40
_F = _H * _HD
_R = 80

_NC, _NS = 2, 16
_NW = _NC * _NS
_CHUNK = 128
_CPW = 80
_EPW = _CPW * _CHUNK
_EPAD = _NW * _EPW
_NACC = 10112

_BLK = 2000


def _expand_attn(a):
    hh, hd = a.shape
    m = (jnp.arange(hh * hd)[:, None] // hd) == jnp.arange(hh)[None, :]
    return jnp.where(m, a.reshape(-1, 1), 0.0).astype(jnp.float32)


def _assemble(h, asv, adv, t1_ref, t2_ref):
    b = h.shape[0]
    nh = asv.shape[1]
    zpad = jnp.zeros((b, 16 - nh), jnp.float32)
    parts = [asv, zpad, h]
    tail = _R - 16 - h.shape[1]
    if tail:
        parts.append(jnp.zeros((b, tail), jnp.float32))
    t1_ref[...] = jnp.concatenate(parts, axis=1)
    t2_ref[...] = jnp.concatenate([adv, zpad], axis=1)


def _prep1_body(x_ref, w_ref, as_ref, ad_ref, t1_ref, t2_ref):
    h = jnp.dot(x_ref[...], w_ref[...], preferred_element_type=jnp.float32)
    asv = jnp.dot(h, as_ref[...], preferred_element_type=jnp.float32)
    adv = jnp.dot(h, ad_ref[...], preferred_element_type=jnp.float32)
    _assemble(h, asv, adv, t1_ref, t2_ref)


def _mid_body(p_ref, b_ref, w_ref, as_ref, ad_ref, t1_ref, t2_ref):
    acc = p_ref[0] + p_ref[1]
    den = acc[:, 0:_H]
    num = acc[:, 16:16 + _F]
    rep = (lax.broadcasted_iota(jnp.int32, (_H, _F), 1) // _HD ==
           lax.broadcasted_iota(jnp.int32, (_H, _F), 0)).astype(jnp.float32)
    denr = jnp.dot(den, rep, preferred_element_type=jnp.float32)
    xv = jnp.maximum(num / (denr + 1e-16) + b_ref[...], 0.0)
    h = jnp.dot(xv, w_ref[...], preferred_element_type=jnp.float32)
    asv = jnp.dot(h, as_ref[...], preferred_element_type=jnp.float32)
    adv = jnp.dot(h, ad_ref[...], preferred_element_type=jnp.float32)
    _assemble(h, asv, adv, t1_ref, t2_ref)


def _final_body(p_ref, b_ref, o_ref):
    acc = p_ref[0] + p_ref[1]
    o_ref[...] = acc[:, 16:16 + _C] / (acc[:, 0:1] + 1e-16) + b_ref[...]


def _prep1(x, w, a_s, a_d):
    return pl.pallas_call(
        _prep1_body,
        grid=(_N // _BLK,),
        in_specs=[pl.BlockSpec((_BLK, _D), lambda i: (i, 0)),
                  pl.BlockSpec((_D, _F), lambda i: (0, 0)),
                  pl.BlockSpec((_F, _H), lambda i: (0, 0)),
                  pl.BlockSpec((_F, _H), lambda i: (0, 0))],
        out_specs=[pl.BlockSpec((_BLK, _R), lambda i: (i, 0)),
                   pl.BlockSpec((_BLK, 16), lambda i: (i, 0))],
        out_shape=[jax.ShapeDtypeStruct((_N, _R), jnp.float32),
                   jax.ShapeDtypeStruct((_N, 16), jnp.float32)],
    )(x, w, a_s, a_d)


def _mid(p, bias, w, a_s, a_d):
    fo = w.shape[1]
    nh = a_s.shape[1]
    return pl.pallas_call(
        _mid_body,
        grid=(_N // _BLK,),
        in_specs=[pl.BlockSpec((2, _BLK, _R), lambda i: (0, i, 0)),
                  pl.BlockSpec((1, _F), lambda i: (0, 0)),
                  pl.BlockSpec((_F, fo), lambda i: (0, 0)),
                  pl.BlockSpec((fo, nh), lambda i: (0, 0)),
                  pl.BlockSpec((fo, nh), lambda i: (0, 0))],
        out_specs=[pl.BlockSpec((_BLK, _R), lambda i: (i, 0)),
                   pl.BlockSpec((_BLK, 16), lambda i: (i, 0))],
        out_shape=[jax.ShapeDtypeStruct((_N, _R), jnp.float32),
                   jax.ShapeDtypeStruct((_N, 16), jnp.float32)],
    )(p, bias, w, a_s, a_d)


def _final(p, bias):
    return pl.pallas_call(
        _final_body,
        grid=(_N // _BLK,),
        in_specs=[pl.BlockSpec((2, _BLK, _R), lambda i: (0, i, 0)),
                  pl.BlockSpec((1, _C), lambda i: (0, 0))],
        out_specs=pl.BlockSpec((_BLK, _C), lambda i: (i, 0)),
        out_shape=jax.ShapeDtypeStruct((_N, _C), jnp.float32),
    )(p, bias)


def _sc_layer_fn(multi):
    mesh = plsc.VectorSubcoreMesh(core_axis_name="c", subcore_axis_name="s",
                                  num_cores=_NC, num_subcores=_NS)

    def body(t1_hbm, t2_hbm, src_hbm, dstg_hbm, dsts_hbm, z_hbm, out_hbm,
             sid_v, dgd_v, did_v, rows_v, t2_v, sem1, sem2, acc):
        cid = lax.axis_index("c")
        sid = lax.axis_index("s")
        wid = sid * _NC + cid
        zrows = _NACC // _NS
        r0 = sid * zrows
        pltpu.sync_copy(z_hbm.at[pl.ds(r0, zrows)], acc.at[pl.ds(r0, zrows)])
        pltpu.sync_copy(src_hbm.at[pl.ds(wid * _CPW, _CPW)], sid_v)
        pltpu.sync_copy(dstg_hbm.at[pl.ds(wid * _CPW, _CPW)], dgd_v)
        pltpu.sync_copy(dsts_hbm.at[pl.ds(wid * _CPW, _CPW)], did_v)
        plsc.subcore_barrier()

        def chunk(j, carry):
            pltpu.async_copy(t1_hbm.at[sid_v.at[j]], rows_v, sem1).wait()
            pltpu.async_copy(t2_hbm.at[dgd_v.at[j]], t2_v, sem2).wait()

            def edge(i, c2):
                av = rows_v[i, pl.ds(0, 16)]
                dv = t2_v[i, :]
                e = av + dv
                e = jnp.maximum(e, e * 0.2)
                ex = jnp.exp(e)
                rows_v[i, pl.ds(0, 16)] = ex
                half = lax.iota(jnp.int32, 16) < 8
                for k in range(4):
                    if multi:
                        exb = jnp.where(half, ex[2 * k], ex[2 * k + 1])
                    else:
                        exb = jnp.where(half, ex[0], ex[0])
                    hv = rows_v[i, pl.ds(16 + 16 * k, 16)]
                    rows_v[i, pl.ds(16 + 16 * k, 16)] = hv * exb
                return c2

            lax.fori_loop(0, _CHUNK, edge, 0, unroll=2)
            pltpu.sync_copy(rows_v, acc.at[did_v.at[j]], add=True)
            return carry

        lax.fori_loop(0, _CPW, chunk, 0)
        plsc.subcore_barrier()
        pltpu.sync_copy(acc.at[pl.ds(r0, zrows)],
                        out_hbm.at[cid, pl.ds(r0, zrows)])

    return pl.kernel(
        body,
        out_type=jax.ShapeDtypeStruct((_NC, _NACC, _R), jnp.float32),
        mesh=mesh,
        compiler_params=pltpu.CompilerParams(use_tc_tiling_on_sc=False),
        scratch_types=[
            pltpu.VMEM((_CPW, _CHUNK), jnp.int32),
            pltpu.VMEM((_CPW, _CHUNK), jnp.int32),
            pltpu.VMEM((_CPW, _CHUNK), jnp.int32),
            pltpu.VMEM((_CHUNK, _R), jnp.float32),
            pltpu.VMEM((_CHUNK, 16), jnp.float32),
            pltpu.SemaphoreType.DMA,
            pltpu.SemaphoreType.DMA,
            pltpu.VMEM_SHARED((_NACC, _R), jnp.float32),
        ],
    )


def kernel(x, edge_index, W1, a1s, a1d, b1, W2, a2s, a2d, b2,
           W3, a3s, a3d, b3, W4, a4s, a4d, b4):
    src = edge_index[0].astype(jnp.int32)
    dst = edge_index[1].astype(jnp.int32)
    npad = _EPAD - _E
    src2d = jnp.concatenate(
        [src, jnp.zeros((npad,), jnp.int32)]).reshape(_EPAD // _CHUNK, _CHUNK)
    dstg2d = jnp.concatenate(
        [dst, jnp.zeros((npad,), jnp.int32)]).reshape(_EPAD // _CHUNK, _CHUNK)
    dsts2d = jnp.concatenate(
        [dst, jnp.full((npad,), _N, jnp.int32)]).reshape(_EPAD // _CHUNK, _CHUNK)
    zrows = jnp.zeros((_NACC, _R), jnp.float32)

    sc_multi = _sc_layer_fn(True)
    sc_single = _sc_layer_fn(False)

    t1, t2 = _prep1(x, W1, _expand_attn(a1s), _expand_attn(a1d))
    p = sc_multi(t1, t2, src2d, dstg2d, dsts2d, zrows)
    t1, t2 = _mid(p, b1.reshape(1, _F), W2, _expand_attn(a2s), _expand_attn(a2d))
    p = sc_multi(t1, t2, src2d, dstg2d, dsts2d, zrows)
    t1, t2 = _mid(p, b2.reshape(1, _F), W3, _expand_attn(a3s), _expand_attn(a3d))
    p = sc_multi(t1, t2, src2d, dstg2d, dsts2d, zrows)
    t1, t2 = _mid(p, b3.reshape(1, _F), W4, _expand_attn(a4s), _expand_attn(a4d))
    p = sc_single(t1, t2, src2d, dstg2d, dsts2d, zrows)
    return _final(p, b4.reshape(1, _C))


# speedup vs baseline: 74.3794x; 1.6979x over previous
import functools

import jax
import jax.numpy as jnp
from jax import lax
from jax.experimental import pallas as pl
from jax.experimental.pallas import tpu as pltpu
from jax.experimental.pallas import tpu_sc as plsc

_N = 10000
_E = 320000
_D = 128
_H = 8
_HD = 8
_C = 40
_F = _H * _HD
_R = 80

_NC, _NS = 2, 16
_NW = _NC * _NS
_CHUNK = 128
_CPW = 80
_EPW = _CPW * _CHUNK
_EPAD = _NW * _EPW
_NACC = 10112

_BLK = 2000


def _expand_attn(a):
    hh, hd = a.shape
    m = (jnp.arange(hh * hd)[:, None] // hd) == jnp.arange(hh)[None, :]
    return jnp.where(m, a.reshape(-1, 1), 0.0).astype(jnp.float32)


def _assemble(h, asv, adv, t1_ref, t2_ref):
    b = h.shape[0]
    nh = asv.shape[1]
    zpad = jnp.zeros((b, 16 - nh), jnp.float32)
    parts = [asv, zpad, h]
    tail = _R - 16 - h.shape[1]
    if tail:
        parts.append(jnp.zeros((b, tail), jnp.float32))
    t1_ref[...] = jnp.concatenate(parts, axis=1)
    t2_ref[...] = jnp.concatenate([adv, zpad], axis=1)


def _prep1_body(x_ref, w_ref, as_ref, ad_ref, t1_ref, t2_ref):
    h = jnp.dot(x_ref[...], w_ref[...], preferred_element_type=jnp.float32)
    asv = jnp.dot(h, as_ref[...], preferred_element_type=jnp.float32)
    adv = jnp.dot(h, ad_ref[...], preferred_element_type=jnp.float32)
    _assemble(h, asv, adv, t1_ref, t2_ref)


def _mid_body(p_ref, b_ref, w_ref, as_ref, ad_ref, t1_ref, t2_ref):
    acc = p_ref[0] + p_ref[1]
    den = acc[:, 0:_H]
    num = acc[:, 16:16 + _F]
    rep = (lax.broadcasted_iota(jnp.int32, (_H, _F), 1) // _HD ==
           lax.broadcasted_iota(jnp.int32, (_H, _F), 0)).astype(jnp.float32)
    denr = jnp.dot(den, rep, preferred_element_type=jnp.float32)
    xv = jnp.maximum(num / (denr + 1e-16) + b_ref[...], 0.0)
    h = jnp.dot(xv, w_ref[...], preferred_element_type=jnp.float32)
    asv = jnp.dot(h, as_ref[...], preferred_element_type=jnp.float32)
    adv = jnp.dot(h, ad_ref[...], preferred_element_type=jnp.float32)
    _assemble(h, asv, adv, t1_ref, t2_ref)


def _final_body(p_ref, b_ref, o_ref):
    acc = p_ref[0] + p_ref[1]
    o_ref[...] = acc[:, 16:16 + _C] / (acc[:, 0:1] + 1e-16) + b_ref[...]


def _prep1(x, w, a_s, a_d):
    return pl.pallas_call(
        _prep1_body,
        grid=(_N // _BLK,),
        in_specs=[pl.BlockSpec((_BLK, _D), lambda i: (i, 0)),
                  pl.BlockSpec((_D, _F), lambda i: (0, 0)),
                  pl.BlockSpec((_F, _H), lambda i: (0, 0)),
                  pl.BlockSpec((_F, _H), lambda i: (0, 0))],
        out_specs=[pl.BlockSpec((_BLK, _R), lambda i: (i, 0)),
                   pl.BlockSpec((_BLK, 16), lambda i: (i, 0))],
        out_shape=[jax.ShapeDtypeStruct((_N, _R), jnp.float32),
                   jax.ShapeDtypeStruct((_N, 16), jnp.float32)],
    )(x, w, a_s, a_d)


def _mid(p, bias, w, a_s, a_d):
    fo = w.shape[1]
    nh = a_s.shape[1]
    return pl.pallas_call(
        _mid_body,
        grid=(_N // _BLK,),
        in_specs=[pl.BlockSpec((2, _BLK, _R), lambda i: (0, i, 0)),
                  pl.BlockSpec((1, _F), lambda i: (0, 0)),
                  pl.BlockSpec((_F, fo), lambda i: (0, 0)),
                  pl.BlockSpec((fo, nh), lambda i: (0, 0)),
                  pl.BlockSpec((fo, nh), lambda i: (0, 0))],
        out_specs=[pl.BlockSpec((_BLK, _R), lambda i: (i, 0)),
                   pl.BlockSpec((_BLK, 16), lambda i: (i, 0))],
        out_shape=[jax.ShapeDtypeStruct((_N, _R), jnp.float32),
                   jax.ShapeDtypeStruct((_N, 16), jnp.float32)],
    )(p, bias, w, a_s, a_d)


def _final(p, bias):
    return pl.pallas_call(
        _final_body,
        grid=(_N // _BLK,),
        in_specs=[pl.BlockSpec((2, _BLK, _R), lambda i: (0, i, 0)),
                  pl.BlockSpec((1, _C), lambda i: (0, 0))],
        out_specs=pl.BlockSpec((_BLK, _C), lambda i: (i, 0)),
        out_shape=jax.ShapeDtypeStruct((_N, _C), jnp.float32),
    )(p, bias)


def _sc_layer_fn(multi):
    mesh = plsc.VectorSubcoreMesh(core_axis_name="c", subcore_axis_name="s",
                                  num_cores=_NC, num_subcores=_NS)

    def body(t1_hbm, t2_hbm, src_hbm, dstg_hbm, dsts_hbm, z_hbm, out_hbm,
             sid_v, dgd_v, did_v, rows_a, rows_b, t2_a, t2_b,
             sem_ga, sem_gb, sem_sa, sem_sb, acc):
        cid = lax.axis_index("c")
        sid = lax.axis_index("s")
        wid = sid * _NC + cid
        zrows = _NACC // _NS
        r0 = sid * zrows
        pltpu.sync_copy(z_hbm.at[pl.ds(r0, zrows)], acc.at[pl.ds(r0, zrows)])
        pltpu.sync_copy(src_hbm.at[pl.ds(wid * _CPW, _CPW)], sid_v)
        pltpu.sync_copy(dstg_hbm.at[pl.ds(wid * _CPW, _CPW)], dgd_v)
        pltpu.sync_copy(dsts_hbm.at[pl.ds(wid * _CPW, _CPW)], did_v)
        plsc.subcore_barrier()

        def fire_gather(j, rows, t2b, sem):
            pltpu.async_copy(t1_hbm.at[sid_v.at[j]], rows, sem)
            pltpu.async_copy(t2_hbm.at[dgd_v.at[j]], t2b, sem)

        def wait_gather(rows, t2b, sem):
            pltpu.make_async_copy(t1_hbm.at[sid_v.at[0]], rows, sem).wait()
            pltpu.make_async_copy(t2_hbm.at[dgd_v.at[0]], t2b, sem).wait()

        def fire_scatter(j, rows, sem):
            pltpu.async_copy(rows, acc.at[did_v.at[j]], sem, add=True)

        def wait_scatter(rows, sem):
            pltpu.make_async_copy(rows, acc.at[did_v.at[0]], sem).wait()

        def compute(rows_v, t2_v):
            def edge(i, c2):
                av = rows_v[i, pl.ds(0, 16)]
                dv = t2_v[i, :]
                e = av + dv
                e = jnp.maximum(e, e * 0.2)
                ex = jnp.exp(e)
                rows_v[i, pl.ds(0, 16)] = ex
                half = lax.iota(jnp.int32, 16) < 8
                for k in range(4):
                    if multi:
                        exb = jnp.where(half, ex[2 * k], ex[2 * k + 1])
                    else:
                        exb = jnp.where(half, ex[0], ex[0])
                    hv = rows_v[i, pl.ds(16 + 16 * k, 16)]
                    rows_v[i, pl.ds(16 + 16 * k, 16)] = hv * exb
                return c2

            lax.fori_loop(0, _CHUNK, edge, 0, unroll=4)

        npair = _CPW // 2
        fire_gather(0, rows_a, t2_a, sem_ga)

        def pair(m, carry):
            ja = 2 * m

            @pl.when(m > 0)
            def _():
                wait_scatter(rows_b, sem_sb)
            fire_gather(ja + 1, rows_b, t2_b, sem_gb)
            wait_gather(rows_a, t2_a, sem_ga)
            compute(rows_a, t2_a)
            fire_scatter(ja, rows_a, sem_sa)

            @pl.when(m < npair - 1)
            def _():
                wait_scatter(rows_a, sem_sa)
                fire_gather(ja + 2, rows_a, t2_a, sem_ga)
            wait_gather(rows_b, t2_b, sem_gb)
            compute(rows_b, t2_b)
            fire_scatter(ja + 1, rows_b, sem_sb)
            return carry

        lax.fori_loop(0, npair, pair, 0)
        wait_scatter(rows_a, sem_sa)
        wait_scatter(rows_b, sem_sb)
        plsc.subcore_barrier()
        pltpu.sync_copy(acc.at[pl.ds(r0, zrows)],
                        out_hbm.at[cid, pl.ds(r0, zrows)])

    return pl.kernel(
        body,
        out_type=jax.ShapeDtypeStruct((_NC, _NACC, _R), jnp.float32),
        mesh=mesh,
        compiler_params=pltpu.CompilerParams(use_tc_tiling_on_sc=False),
        scratch_types=[
            pltpu.VMEM((_CPW, _CHUNK), jnp.int32),
            pltpu.VMEM((_CPW, _CHUNK), jnp.int32),
            pltpu.VMEM((_CPW, _CHUNK), jnp.int32),
            pltpu.VMEM((_CHUNK, _R), jnp.float32),
            pltpu.VMEM((_CHUNK, _R), jnp.float32),
            pltpu.VMEM((_CHUNK, 16), jnp.float32),
            pltpu.VMEM((_CHUNK, 16), jnp.float32),
            pltpu.SemaphoreType.DMA,
            pltpu.SemaphoreType.DMA,
            pltpu.SemaphoreType.DMA,
            pltpu.SemaphoreType.DMA,
            pltpu.VMEM_SHARED((_NACC, _R), jnp.float32),
        ],
    )


def kernel(x, edge_index, W1, a1s, a1d, b1, W2, a2s, a2d, b2,
           W3, a3s, a3d, b3, W4, a4s, a4d, b4):
    src = edge_index[0].astype(jnp.int32)
    dst = edge_index[1].astype(jnp.int32)
    npad = _EPAD - _E
    src2d = jnp.concatenate(
        [src, jnp.zeros((npad,), jnp.int32)]).reshape(_EPAD // _CHUNK, _CHUNK)
    dstg2d = jnp.concatenate(
        [dst, jnp.zeros((npad,), jnp.int32)]).reshape(_EPAD // _CHUNK, _CHUNK)
    dsts2d = jnp.concatenate(
        [dst, jnp.full((npad,), _N, jnp.int32)]).reshape(_EPAD // _CHUNK, _CHUNK)
    zrows = jnp.zeros((_NACC, _R), jnp.float32)

    sc_multi = _sc_layer_fn(True)
    sc_single = _sc_layer_fn(False)

    t1, t2 = _prep1(x, W1, _expand_attn(a1s), _expand_attn(a1d))
    p = sc_multi(t1, t2, src2d, dstg2d, dsts2d, zrows)
    t1, t2 = _mid(p, b1.reshape(1, _F), W2, _expand_attn(a2s), _expand_attn(a2d))
    p = sc_multi(t1, t2, src2d, dstg2d, dsts2d, zrows)
    t1, t2 = _mid(p, b2.reshape(1, _F), W3, _expand_attn(a3s), _expand_attn(a3d))
    p = sc_multi(t1, t2, src2d, dstg2d, dsts2d, zrows)
    t1, t2 = _mid(p, b3.reshape(1, _F), W4, _expand_attn(a4s), _expand_attn(a4d))
    p = sc_single(t1, t2, src2d, dstg2d, dsts2d, zrows)
    return _final(p, b4.reshape(1, _C))
